# baseline (speedup 1.0000x reference)
import jax
import jax.numpy as jnp
from jax import lax
from jax.experimental import pallas as pl
from jax.experimental.pallas import tpu as pltpu
from jax.experimental.pallas import tpu_sc as plsc

_N_LM = 68
_LANES = 16
_PAD = 80
_CHUNKS = _PAD // _LANES
_TAIL = _N_LM - (_CHUNKS - 1) * _LANES


def _sc_body(points_hbm, lm_hbm, out_hbm, lm_v, rows_e, rows_o, out_v, sem):
    cid = lax.axis_index("c")
    sid = lax.axis_index("s")

    @pl.when(jnp.logical_and(cid == 0, sid == 0))
    def _():
        pltpu.sync_copy(lm_hbm, lm_v)

        copies = []
        for c in range(_CHUNKS):
            idx = lm_v[pl.ds(c * _LANES, _LANES)]
            even = idx * 2
            odd = even + 1
            copies.append(pltpu.async_copy(
                points_hbm.at[even], rows_e.at[pl.ds(c * _LANES, _LANES)],
                sem))
            copies.append(pltpu.async_copy(
                points_hbm.at[odd], rows_o.at[pl.ds(c * _LANES, _LANES)],
                sem))
        for cp in copies:
            cp.wait()

        acc = jnp.zeros((_LANES,), jnp.float32)
        for c in range(_CHUNKS):
            ve = rows_e[pl.ds(c * _LANES, _LANES)]
            vo = rows_o[pl.ds(c * _LANES, _LANES)]
            s = ve * ve + vo * vo
            if c == _CHUNKS - 1:
                s = jnp.where(lax.iota(jnp.int32, _LANES) < _TAIL, s, 0.0)
            acc = acc + s

        lanes = lax.iota(jnp.int32, _LANES)
        dnums = lax.GatherDimensionNumbers(
            offset_dims=(), collapsed_slice_dims=(0,), start_index_map=(0,))
        t = acc
        for s in (8, 4, 2, 1):
            t = t + lax.gather(
                t, (lanes ^ s)[:, None], dnums, (1,),
                mode=lax.GatherScatterMode.PROMISE_IN_BOUNDS)

        bits = lax.bitcast_convert_type(t, jnp.int32)
        y = lax.bitcast_convert_type(
            jnp.int32(0x5F3759DF) - (bits >> 1), jnp.float32)
        half = jnp.float32(0.5) * t
        for _ in range(4):
            y = y * (jnp.float32(1.5) - half * y * y)
        out_v[...] = t * y
        pltpu.sync_copy(out_v, out_hbm)


def kernel(pointsUV, landmarks):
    flat = pointsUV.reshape(-1)
    lm = jnp.zeros((_PAD,), jnp.int32).at[:_N_LM].set(
        landmarks.astype(jnp.int32))
    f = pl.kernel(
        _sc_body,
        out_type=jax.ShapeDtypeStruct((_LANES,), jnp.float32),
        mesh=plsc.VectorSubcoreMesh(core_axis_name="c", subcore_axis_name="s"),
        scratch_types=[
            pltpu.VMEM((_PAD,), jnp.int32),
            pltpu.VMEM((_PAD,), jnp.float32),
            pltpu.VMEM((_PAD,), jnp.float32),
            pltpu.VMEM((_LANES,), jnp.float32),
            pltpu.SemaphoreType.DMA,
        ],
    )
    return f(flat, lm)[0]


# speedup vs baseline: 1.0090x; 1.0090x over previous
import jax
import jax.numpy as jnp
from jax import lax
from jax.experimental import pallas as pl
from jax.experimental.pallas import tpu as pltpu
from jax.experimental.pallas import tpu_sc as plsc

_N_LM = 68
_LANES = 16
_PAD = 80
_CHUNKS = _PAD // _LANES
_TAIL = _N_LM - (_CHUNKS - 1) * _LANES


def _sc_body(points_hbm, lm_hbm, out_hbm, lm_v, rows_e, rows_o, out_v, sem):
    cid = lax.axis_index("c")
    sid = lax.axis_index("s")

    @pl.when(jnp.logical_and(cid == 0, sid == 0))
    def _():
        pltpu.sync_copy(lm_hbm, lm_v.at[pl.ds(0, _N_LM)])

        copies = []
        for c in range(_CHUNKS):
            idx = lm_v[pl.ds(c * _LANES, _LANES)]
            if c == _CHUNKS - 1:
                idx = jnp.where(lax.iota(jnp.int32, _LANES) < _TAIL, idx, 0)
            even = idx * 2
            odd = even + 1
            copies.append(pltpu.async_copy(
                points_hbm.at[even], rows_e.at[pl.ds(c * _LANES, _LANES)],
                sem))
            copies.append(pltpu.async_copy(
                points_hbm.at[odd], rows_o.at[pl.ds(c * _LANES, _LANES)],
                sem))
        for cp in copies:
            cp.wait()

        acc = jnp.zeros((_LANES,), jnp.float32)
        for c in range(_CHUNKS):
            ve = rows_e[pl.ds(c * _LANES, _LANES)]
            vo = rows_o[pl.ds(c * _LANES, _LANES)]
            s = ve * ve + vo * vo
            if c == _CHUNKS - 1:
                s = jnp.where(lax.iota(jnp.int32, _LANES) < _TAIL, s, 0.0)
            acc = acc + s

        lanes = lax.iota(jnp.int32, _LANES)
        dnums = lax.GatherDimensionNumbers(
            offset_dims=(), collapsed_slice_dims=(0,), start_index_map=(0,))
        t = acc
        for s in (8, 4, 2, 1):
            t = t + lax.gather(
                t, (lanes ^ s)[:, None], dnums, (1,),
                mode=lax.GatherScatterMode.PROMISE_IN_BOUNDS)

        bits = lax.bitcast_convert_type(t, jnp.int32)
        y = lax.bitcast_convert_type(
            jnp.int32(0x5F3759DF) - (bits >> 1), jnp.float32)
        half = jnp.float32(0.5) * t
        for _ in range(4):
            y = y * (jnp.float32(1.5) - half * y * y)
        out_v[...] = t * y
        pltpu.sync_copy(out_v.at[pl.ds(0, 1)], out_hbm)


def kernel(pointsUV, landmarks):
    flat = pointsUV.reshape(-1)
    lm = landmarks.astype(jnp.int32)
    f = pl.kernel(
        _sc_body,
        out_type=jax.ShapeDtypeStruct((1,), jnp.float32),
        mesh=plsc.VectorSubcoreMesh(core_axis_name="c", subcore_axis_name="s"),
        scratch_types=[
            pltpu.VMEM((_PAD,), jnp.int32),
            pltpu.VMEM((_PAD,), jnp.float32),
            pltpu.VMEM((_PAD,), jnp.float32),
            pltpu.VMEM((_LANES,), jnp.float32),
            pltpu.SemaphoreType.DMA,
        ],
    )
    return f(flat, lm)[0]


# speedup vs baseline: 1.0283x; 1.0192x over previous
import jax
import jax.numpy as jnp
from jax import lax
from jax.experimental import pallas as pl
from jax.experimental.pallas import tpu as pltpu
from jax.experimental.pallas import tpu_sc as plsc

_N_LM = 68
_LANES = 16
_PAD = 80
_CHUNKS = _PAD // _LANES
_TAIL = _N_LM - (_CHUNKS - 1) * _LANES


def _sc_body(points_hbm, lm_hbm, out_hbm, lm_v, rows_e, rows_o, out_v, sem):
    cid = lax.axis_index("c")
    sid = lax.axis_index("s")

    @pl.when(jnp.logical_and(cid == 0, sid == 0))
    def _():
        pltpu.sync_copy(lm_hbm, lm_v.at[pl.ds(0, _N_LM)])

        copies = []
        for c in range(_CHUNKS):
            idx = lm_v[pl.ds(c * _LANES, _LANES)]
            if c == _CHUNKS - 1:
                idx = jnp.where(lax.iota(jnp.int32, _LANES) < _TAIL, idx, 0)
            even = idx * 2
            odd = even + 1
            copies.append(pltpu.async_copy(
                points_hbm.at[even], rows_e.at[pl.ds(c * _LANES, _LANES)],
                sem))
            copies.append(pltpu.async_copy(
                points_hbm.at[odd], rows_o.at[pl.ds(c * _LANES, _LANES)],
                sem))
        for cp in copies:
            cp.wait()

        acc = jnp.zeros((_LANES,), jnp.float32)
        for c in range(_CHUNKS):
            ve = rows_e[pl.ds(c * _LANES, _LANES)]
            vo = rows_o[pl.ds(c * _LANES, _LANES)]
            s = ve * ve + vo * vo
            if c == _CHUNKS - 1:
                s = jnp.where(lax.iota(jnp.int32, _LANES) < _TAIL, s, 0.0)
            acc = acc + s

        lanes = lax.iota(jnp.int32, _LANES)
        dnums = lax.GatherDimensionNumbers(
            offset_dims=(), collapsed_slice_dims=(0,), start_index_map=(0,))
        t = acc
        for s in (8, 4, 2, 1):
            t = t + lax.gather(
                t, (lanes ^ s)[:, None], dnums, (1,),
                mode=lax.GatherScatterMode.PROMISE_IN_BOUNDS)

        bits = lax.bitcast_convert_type(t, jnp.int32)
        y = lax.bitcast_convert_type(
            jnp.int32(0x5F3759DF) - (bits >> 1), jnp.float32)
        half = jnp.float32(0.5) * t
        for _ in range(4):
            y = y * (jnp.float32(1.5) - half * y * y)
        out_v[...] = t * y
        pltpu.sync_copy(out_v.at[pl.ds(0, 1)], out_hbm)


def kernel(pointsUV, landmarks):
    flat = pointsUV.reshape(-1)
    lm = landmarks.astype(jnp.int32)
    f = pl.kernel(
        _sc_body,
        out_type=jax.ShapeDtypeStruct((1,), jnp.float32),
        mesh=plsc.VectorSubcoreMesh(core_axis_name="c", subcore_axis_name="s",
                                    num_cores=1, num_subcores=1),
        scratch_types=[
            pltpu.VMEM((_PAD,), jnp.int32),
            pltpu.VMEM((_PAD,), jnp.float32),
            pltpu.VMEM((_PAD,), jnp.float32),
            pltpu.VMEM((_LANES,), jnp.float32),
            pltpu.SemaphoreType.DMA,
        ],
    )
    return f(flat, lm)[0]
